# Initial kernel scaffold; baseline (speedup 1.0000x reference)
import jax
import jax.numpy as jnp
from jax import lax
from jax.experimental import pallas as pl
from jax.experimental.pallas import tpu as pltpu
from jax.experimental.pallas import tpu_sc as plsc

B, S, L, V, D = 1024, 200, 8, 100000, 128
NC, NS = 2, 16
NW = NC * NS
BPW = B // NW
LANES = 16
C0, C1 = 128, S - 128


def _body(table_hbm, ids_hbm, starts_hbm, fusion_hbm, out_hbm,
          ids_v, starts_v, fusion_v, rows_v, sem):
    wid = lax.axis_index("s") * NC + lax.axis_index("c")
    base = wid * BPW
    pltpu.sync_copy(ids_hbm.at[pl.ds(base, BPW)], ids_v)
    pltpu.sync_copy(starts_hbm.at[pl.ds(base, BPW)], starts_v)
    pltpu.sync_copy(fusion_hbm.at[pl.ds(base, BPW)], fusion_v)

    col_iota = lax.iota(jnp.int32, LANES)

    def step(j, carry):
        cp1 = pltpu.async_copy(table_hbm.at[ids_v.at[j, pl.ds(0, C0)]],
                               rows_v.at[pl.ds(0, C0)], sem)
        cp2 = pltpu.async_copy(table_hbm.at[ids_v.at[j, pl.ds(C0, C1)]],
                               rows_v.at[pl.ds(C0, C1)], sem)
        cp1.wait()
        cp2.wait()
        jvec = jnp.full((LANES,), 0, jnp.int32) + j
        start_vec = plsc.load_gather(starts_v, [jvec])
        for l in range(L):
            row_idx = start_vec + l
            for c in range(D // LANES):
                x = fusion_v[j, l, pl.ds(c * LANES, LANES)]
                plsc.addupdate_scatter(rows_v, [row_idx, col_iota + c * LANES], x)
        pltpu.sync_copy(rows_v, out_hbm.at[base + j])
        return carry

    lax.fori_loop(0, BPW, step, 0)


_mesh = plsc.VectorSubcoreMesh(core_axis_name="c", subcore_axis_name="s")

_sc_call = pl.kernel(
    _body,
    out_type=jax.ShapeDtypeStruct((B, S, D), jnp.float32),
    mesh=_mesh,
    scratch_types=[
        pltpu.VMEM((BPW, S), jnp.int32),
        pltpu.VMEM((BPW,), jnp.int32),
        pltpu.VMEM((BPW, L, D), jnp.float32),
        pltpu.VMEM((S, D), jnp.float32),
        pltpu.SemaphoreType.DMA,
    ],
)


def kernel(embed_table, input_ids, starts, fusion):
    return _sc_call(embed_table,
                    input_ids.astype(jnp.int32),
                    starts.astype(jnp.int32),
                    fusion)


# speedup vs baseline: 6.9317x; 6.9317x over previous
import jax
import jax.numpy as jnp
from jax import lax
from jax.experimental import pallas as pl
from jax.experimental.pallas import tpu as pltpu
from jax.experimental.pallas import tpu_sc as plsc

B, S, L, V, D = 1024, 200, 8, 100000, 128
NC, NS = 2, 16
NW = NC * NS
BPW = B // NW
LANES = 16
C0, C1 = 128, S - 128


def _body(table_hbm, ids_hbm, starts_hbm, fusion_hbm, out_hbm,
          ids_v, starts_v, fusion_v, rows_v, sem):
    wid = lax.axis_index("s") * NC + lax.axis_index("c")
    base = wid * BPW
    pltpu.sync_copy(ids_hbm.at[pl.ds(base, BPW)], ids_v)
    pltpu.sync_copy(starts_hbm.at[pl.ds(base, BPW)], starts_v)
    pltpu.sync_copy(fusion_hbm.at[pl.ds(base, BPW)], fusion_v)

    col_iota = lax.iota(jnp.int32, LANES)

    def step(j, carry):
        cp1 = pltpu.async_copy(table_hbm.at[ids_v.at[j, pl.ds(0, C0)]],
                               rows_v.at[pl.ds(0, C0)], sem)
        cp2 = pltpu.async_copy(table_hbm.at[ids_v.at[j, pl.ds(C0, C1)]],
                               rows_v.at[pl.ds(C0, C1)], sem)
        cp1.wait()
        cp2.wait()
        jvec = jnp.full((LANES,), 0, jnp.int32) + j
        start_vec = plsc.load_gather(starts_v, [jvec])
        for l in range(L):
            row_idx = start_vec + l
            for c in range(D // LANES):
                x = fusion_v[j, l, pl.ds(c * LANES, LANES)]
                plsc.addupdate_scatter(rows_v, [row_idx, col_iota + c * LANES], x)
        pltpu.sync_copy(rows_v, out_hbm.at[base + j])
        return carry

    lax.fori_loop(0, BPW, step, 0)


_mesh = plsc.VectorSubcoreMesh(core_axis_name="c", subcore_axis_name="s")

_sc_call = pl.kernel(
    _body,
    out_type=jax.ShapeDtypeStruct((B, S, D), jnp.float32),
    mesh=_mesh,
    compiler_params=pltpu.CompilerParams(needs_layout_passes=False),
    scratch_types=[
        pltpu.VMEM((BPW, S), jnp.int32),
        pltpu.VMEM((BPW,), jnp.int32),
        pltpu.VMEM((BPW, L, D), jnp.float32),
        pltpu.VMEM((S, D), jnp.float32),
        pltpu.SemaphoreType.DMA,
    ],
)


def kernel(embed_table, input_ids, starts, fusion):
    return _sc_call(embed_table,
                    input_ids.astype(jnp.int32),
                    starts.astype(jnp.int32),
                    fusion)


# speedup vs baseline: 8.9102x; 1.2854x over previous
import jax
import jax.numpy as jnp
from jax import lax
from jax.experimental import pallas as pl
from jax.experimental.pallas import tpu as pltpu
from jax.experimental.pallas import tpu_sc as plsc

B, S, L, V, D = 1024, 200, 8, 100000, 128
NC, NS = 2, 16
NW = NC * NS
BPW = B // NW
LANES = 16
C0, C1 = 128, S - 128


def _body(table_hbm, ids_hbm, starts_hbm, fusion_hbm, out_hbm,
          ids_v, starts_v, fusion_v, rows_v, sem_g, sem_w):
    wid = lax.axis_index("s") * NC + lax.axis_index("c")
    base = wid * BPW
    pltpu.sync_copy(ids_hbm.at[pl.ds(base, BPW)], ids_v)
    pltpu.sync_copy(starts_hbm.at[pl.ds(base, BPW)], starts_v)
    pltpu.sync_copy(fusion_hbm.at[pl.ds(base, BPW)], fusion_v)

    col_iota = lax.iota(jnp.int32, LANES)
    zeros16 = jnp.full((LANES,), 0, jnp.int32)

    def start_gather(j, k):
        pltpu.async_copy(table_hbm.at[ids_v.at[j, pl.ds(0, C0)]],
                         rows_v.at[k, pl.ds(0, C0)], sem_g.at[k])
        pltpu.async_copy(table_hbm.at[ids_v.at[j, pl.ds(C0, C1)]],
                         rows_v.at[k, pl.ds(C0, C1)], sem_g.at[k])

    def wait_gather(k):
        pltpu.make_async_copy(table_hbm.at[pl.ds(0, C0)],
                              rows_v.at[k, pl.ds(0, C0)], sem_g.at[k]).wait()
        pltpu.make_async_copy(table_hbm.at[pl.ds(0, C1)],
                              rows_v.at[k, pl.ds(C0, C1)], sem_g.at[k]).wait()

    def wait_write(k):
        pltpu.make_async_copy(rows_v.at[k], out_hbm.at[0], sem_w.at[k]).wait()

    start_gather(0, 0)

    def step(j, carry):
        k = j & 1
        kn = 1 - k

        @pl.when(j >= 1)
        def _():
            wait_write(kn)

        @pl.when(j + 1 < BPW)
        def _():
            start_gather(j + 1, kn)

        wait_gather(k)

        kvec = zeros16 + k
        jvec = zeros16 + j
        start_vec = plsc.load_gather(starts_v, [jvec])
        for l in range(L):
            row_idx = start_vec + l
            for c in range(D // LANES):
                x = fusion_v[j, l, pl.ds(c * LANES, LANES)]
                plsc.addupdate_scatter(
                    rows_v, [kvec, row_idx, col_iota + c * LANES], x)

        pltpu.async_copy(rows_v.at[k], out_hbm.at[base + j], sem_w.at[k])
        return carry

    lax.fori_loop(0, BPW, step, 0)
    wait_write((BPW - 1) & 1)


_mesh = plsc.VectorSubcoreMesh(core_axis_name="c", subcore_axis_name="s")

_sc_call = pl.kernel(
    _body,
    out_type=jax.ShapeDtypeStruct((B, S, D), jnp.float32),
    mesh=_mesh,
    compiler_params=pltpu.CompilerParams(needs_layout_passes=False),
    scratch_types=[
        pltpu.VMEM((BPW, S), jnp.int32),
        pltpu.VMEM((BPW,), jnp.int32),
        pltpu.VMEM((BPW, L, D), jnp.float32),
        pltpu.VMEM((2, S, D), jnp.float32),
        pltpu.SemaphoreType.DMA((2,)),
        pltpu.SemaphoreType.DMA((2,)),
    ],
)


def kernel(embed_table, input_ids, starts, fusion):
    return _sc_call(embed_table,
                    input_ids.astype(jnp.int32),
                    starts.astype(jnp.int32),
                    fusion)


# speedup vs baseline: 8.9482x; 1.0043x over previous
import jax
import jax.numpy as jnp
from jax import lax
from jax.experimental import pallas as pl
from jax.experimental.pallas import tpu as pltpu
from jax.experimental.pallas import tpu_sc as plsc

B, S, L, V, D = 1024, 200, 8, 100000, 128
NC, NS = 2, 16
NW = NC * NS
BPW = B // NW
LANES = 16
C0, C1 = 128, S - 128
NBUF = 3


def _body(table_hbm, ids_hbm, starts_hbm, fusion_hbm, out_hbm,
          ids_v, starts_v, fusion_v, rows_v, sem_g, sem_w):
    wid = lax.axis_index("s") * NC + lax.axis_index("c")
    base = wid * BPW
    pltpu.sync_copy(ids_hbm.at[pl.ds(base, BPW)], ids_v)
    pltpu.sync_copy(starts_hbm.at[pl.ds(base, BPW)], starts_v)
    pltpu.sync_copy(fusion_hbm.at[pl.ds(base, BPW)], fusion_v)

    col_iota = lax.iota(jnp.int32, LANES)
    zeros16 = jnp.full((LANES,), 0, jnp.int32)

    def start_gather(j, k):
        pltpu.async_copy(table_hbm.at[ids_v.at[j, pl.ds(0, C0)]],
                         rows_v.at[k, pl.ds(0, C0)], sem_g.at[k])
        pltpu.async_copy(table_hbm.at[ids_v.at[j, pl.ds(C0, C1)]],
                         rows_v.at[k, pl.ds(C0, C1)], sem_g.at[k])

    def wait_gather(k):
        pltpu.make_async_copy(table_hbm.at[pl.ds(0, C0)],
                              rows_v.at[k, pl.ds(0, C0)], sem_g.at[k]).wait()
        pltpu.make_async_copy(table_hbm.at[pl.ds(0, C1)],
                              rows_v.at[k, pl.ds(C0, C1)], sem_g.at[k]).wait()

    def wait_write(k):
        pltpu.make_async_copy(rows_v.at[k], out_hbm.at[0], sem_w.at[k]).wait()

    start_gather(0, 0)
    start_gather(1, 1)

    def step(j, carry):
        k = lax.rem(j, NBUF)

        @pl.when(j + 2 < BPW)
        def _():
            k2 = lax.rem(j + 2, NBUF)

            @pl.when(j >= 1)
            def _():
                wait_write(k2)

            start_gather(j + 2, k2)

        wait_gather(k)

        kvec = zeros16 + k
        jvec = zeros16 + j
        start_vec = plsc.load_gather(starts_v, [jvec])
        for l in range(L):
            row_idx = start_vec + l
            for c in range(D // LANES):
                x = fusion_v[j, l, pl.ds(c * LANES, LANES)]
                plsc.addupdate_scatter(
                    rows_v, [kvec, row_idx, col_iota + c * LANES], x)

        pltpu.async_copy(rows_v.at[k], out_hbm.at[base + j], sem_w.at[k])
        return carry

    lax.fori_loop(0, BPW, step, 0)
    for t in range(NBUF):
        wait_write(lax.rem(jnp.int32(BPW - NBUF + t), NBUF))


_mesh = plsc.VectorSubcoreMesh(core_axis_name="c", subcore_axis_name="s")

_sc_call = pl.kernel(
    _body,
    out_type=jax.ShapeDtypeStruct((B, S, D), jnp.float32),
    mesh=_mesh,
    compiler_params=pltpu.CompilerParams(needs_layout_passes=False),
    scratch_types=[
        pltpu.VMEM((BPW, S), jnp.int32),
        pltpu.VMEM((BPW,), jnp.int32),
        pltpu.VMEM((BPW, L, D), jnp.float32),
        pltpu.VMEM((NBUF, S, D), jnp.float32),
        pltpu.SemaphoreType.DMA((NBUF,)),
        pltpu.SemaphoreType.DMA((NBUF,)),
    ],
)


def kernel(embed_table, input_ids, starts, fusion):
    return _sc_call(embed_table,
                    input_ids.astype(jnp.int32),
                    starts.astype(jnp.int32),
                    fusion)


# speedup vs baseline: 9.0449x; 1.0108x over previous
import jax
import jax.numpy as jnp
from jax import lax
from jax.experimental import pallas as pl
from jax.experimental.pallas import tpu as pltpu
from jax.experimental.pallas import tpu_sc as plsc

B, S, L, V, D = 1024, 200, 8, 100000, 128
NC, NS = 2, 16
NW = NC * NS
BPW = B // NW
LANES = 16
C0, C1 = 128, S - 128
NBUF = 4
PF = 2


def _body(table_hbm, ids_hbm, starts_hbm, fusion_hbm, out_hbm,
          ids_v, starts_v, fus_v, rows_v, sem_g, sem_w):
    wid = lax.axis_index("s") * NC + lax.axis_index("c")
    base = wid * BPW
    pltpu.sync_copy(ids_hbm.at[pl.ds(base, BPW)], ids_v)
    pltpu.sync_copy(starts_hbm.at[pl.ds(base, BPW)], starts_v)

    col_iota = lax.iota(jnp.int32, LANES)
    zeros16 = jnp.full((LANES,), 0, jnp.int32)

    def start_gather(j, k):
        pltpu.async_copy(table_hbm.at[ids_v.at[j, pl.ds(0, C0)]],
                         rows_v.at[k, pl.ds(0, C0)], sem_g.at[k])
        pltpu.async_copy(table_hbm.at[ids_v.at[j, pl.ds(C0, C1)]],
                         rows_v.at[k, pl.ds(C0, C1)], sem_g.at[k])
        pltpu.async_copy(fusion_hbm.at[base + j], fus_v.at[k], sem_g.at[k])

    def wait_gather(k):
        pltpu.make_async_copy(table_hbm.at[pl.ds(0, C0)],
                              rows_v.at[k, pl.ds(0, C0)], sem_g.at[k]).wait()
        pltpu.make_async_copy(table_hbm.at[pl.ds(0, C1)],
                              rows_v.at[k, pl.ds(C0, C1)], sem_g.at[k]).wait()
        pltpu.make_async_copy(fusion_hbm.at[0], fus_v.at[k], sem_g.at[k]).wait()

    def wait_write(k):
        pltpu.make_async_copy(rows_v.at[k], out_hbm.at[0], sem_w.at[k]).wait()

    for p in range(PF):
        start_gather(p, p)

    def step(j, carry):
        k = lax.rem(j, NBUF)

        @pl.when(j + PF < BPW)
        def _():
            k2 = lax.rem(j + PF, NBUF)

            @pl.when(j + PF >= NBUF)
            def _():
                wait_write(k2)

            start_gather(j + PF, k2)

        wait_gather(k)

        kvec = zeros16 + k
        jvec = zeros16 + j
        start_vec = plsc.load_gather(starts_v, [jvec])
        for l in range(L):
            row_idx = start_vec + l
            for c in range(D // LANES):
                x = fus_v[k, l, pl.ds(c * LANES, LANES)]
                plsc.addupdate_scatter(
                    rows_v, [kvec, row_idx, col_iota + c * LANES], x)

        pltpu.async_copy(rows_v.at[k], out_hbm.at[base + j], sem_w.at[k])
        return carry

    lax.fori_loop(0, BPW, step, 0)
    for t in range(NBUF):
        wait_write(lax.rem(jnp.int32(BPW - NBUF + t), NBUF))


_mesh = plsc.VectorSubcoreMesh(core_axis_name="c", subcore_axis_name="s")

_sc_call = pl.kernel(
    _body,
    out_type=jax.ShapeDtypeStruct((B, S, D), jnp.float32),
    mesh=_mesh,
    compiler_params=pltpu.CompilerParams(needs_layout_passes=False),
    scratch_types=[
        pltpu.VMEM((BPW, S), jnp.int32),
        pltpu.VMEM((BPW,), jnp.int32),
        pltpu.VMEM((NBUF, L, D), jnp.float32),
        pltpu.VMEM((NBUF, S, D), jnp.float32),
        pltpu.SemaphoreType.DMA((NBUF,)),
        pltpu.SemaphoreType.DMA((NBUF,)),
    ],
)


def kernel(embed_table, input_ids, starts, fusion):
    return _sc_call(embed_table,
                    input_ids.astype(jnp.int32),
                    starts.astype(jnp.int32),
                    fusion)
